# baseline (speedup 1.0000x reference)
import functools
import jax
import jax.numpy as jnp
from jax import lax
from jax.experimental import pallas as pl
from jax.experimental.pallas import tpu as pltpu
from jax.experimental.pallas import tpu_sc as plsc

N = 10000
D = 128
E = 320000

NC = 2
NS = 16
NW = NC * NS
CHUNK = 64
E_PAD = 327680
EPW = E_PAD // NW
N_CHUNKS = EPW // CHUNK
N_PAD = 10240
ROWS_PER_TILE = N_PAD // NS

_sc_mesh = plsc.VectorSubcoreMesh(core_axis_name="c", subcore_axis_name="s")

NBUF = 4
GROUPS = N_CHUNKS // NBUF

_scratch = [pltpu.VMEM((EPW,), jnp.int32)]
for _ in range(NBUF):
    _scratch.append(pltpu.VMEM((CHUNK,), jnp.int32))
    _scratch.append(pltpu.VMEM((CHUNK, D), jnp.float32))
_scratch.append(pltpu.VMEM_SHARED((N_PAD, D), jnp.float32))
_scratch.extend([pltpu.SemaphoreType.DMA] * (3 * NBUF + 1))


@functools.partial(
    pl.kernel,
    out_type=jax.ShapeDtypeStruct((NC, N_PAD, D), jnp.float32),
    mesh=_sc_mesh,
    scratch_types=_scratch,
)
def _sc_segment_sum(g_hbm, src_hbm, dst_hbm, zeros_hbm, out_hbm, *scr):
    srcv = scr[0]
    dstb = [scr[1 + 2 * b] for b in range(NBUF)]
    rows = [scr[2 + 2 * b] for b in range(NBUF)]
    acc = scr[1 + 2 * NBUF]
    sems = scr[2 + 2 * NBUF:]
    dsem = sems[0:NBUF]
    gsem = sems[NBUF:2 * NBUF]
    ssem = sems[2 * NBUF:3 * NBUF]
    isem = sems[3 * NBUF]
    c = lax.axis_index("c")
    s = lax.axis_index("s")
    wid = s * NC + c
    base = wid * EPW

    pltpu.async_copy(src_hbm.at[pl.ds(base, EPW)], srcv, isem)
    for b in range(NBUF):
        pltpu.async_copy(dst_hbm.at[pl.ds(base + b * CHUNK, CHUNK)],
                         dstb[b], dsem[b])
    pltpu.sync_copy(zeros_hbm, acc.at[pl.ds(s * ROWS_PER_TILE, ROWS_PER_TILE)])
    plsc.subcore_barrier()

    pltpu.make_async_copy(src_hbm.at[pl.ds(base, EPW)], srcv, isem).wait()
    for b in range(NBUF):
        pltpu.async_copy(g_hbm.at[srcv.at[pl.ds(b * CHUNK, CHUNK)]],
                         rows[b], gsem[b])

    def _group(base_ch, refill):
        for b in range(NBUF):
            pltpu.make_async_copy(g_hbm.at[srcv.at[pl.ds(b * CHUNK, CHUNK)]],
                                  rows[b], gsem[b]).wait()
            pltpu.make_async_copy(dst_hbm.at[pl.ds(base, CHUNK)], dstb[b],
                                  dsem[b]).wait()
            pltpu.async_copy(rows[b], acc.at[dstb[b]], ssem[b], add=True)
        for b in range(NBUF):
            pltpu.make_async_copy(rows[b], acc.at[dstb[b]], ssem[b]).wait()
            if refill:
                nxt = base_ch + NBUF + b
                pltpu.async_copy(dst_hbm.at[pl.ds(base + nxt * CHUNK, CHUNK)],
                                 dstb[b], dsem[b])
                pltpu.async_copy(g_hbm.at[srcv.at[pl.ds(nxt * CHUNK, CHUNK)]],
                                 rows[b], gsem[b])

    @pl.loop(0, GROUPS - 1)
    def _(grp):
        _group(grp * NBUF, refill=True)

    _group((GROUPS - 1) * NBUF, refill=False)

    plsc.subcore_barrier()

    pltpu.sync_copy(acc.at[pl.ds(s * ROWS_PER_TILE, ROWS_PER_TILE)],
                    out_hbm.at[c, pl.ds(s * ROWS_PER_TILE, ROWS_PER_TILE)])


ROW_BLK = 2000


def _tc_head(h, wlT, wcT, bias):
    def body(h_ref, wl_ref, wc_ref, b_ref, g_ref, d_ref):
        hb = h_ref[...]
        g_ref[...] = jnp.dot(hb, wl_ref[...], preferred_element_type=jnp.float32)
        d_ref[...] = jnp.dot(hb, wc_ref[...], preferred_element_type=jnp.float32) + b_ref[...]

    return pl.pallas_call(
        body,
        grid=(N // ROW_BLK,),
        in_specs=[
            pl.BlockSpec((ROW_BLK, D), lambda i: (i, 0)),
            pl.BlockSpec((D, D), lambda i: (0, 0)),
            pl.BlockSpec((D, D), lambda i: (0, 0)),
            pl.BlockSpec((1, D), lambda i: (0, 0)),
        ],
        out_specs=[
            pl.BlockSpec((ROW_BLK, D), lambda i: (i, 0)),
            pl.BlockSpec((ROW_BLK, D), lambda i: (i, 0)),
        ],
        out_shape=[
            jax.ShapeDtypeStruct((N, D), jnp.float32),
            jax.ShapeDtypeStruct((N, D), jnp.float32),
        ],
    )(h, wlT, wcT, bias)


def _tc_mid(p, d, wlT, wcT, bias):
    def body(p_ref, d_ref, wl_ref, wc_ref, b_ref, g_ref, d2_ref):
        hb = jnp.maximum(p_ref[0] + p_ref[1] + d_ref[...], 0.0)
        g_ref[...] = jnp.dot(hb, wl_ref[...], preferred_element_type=jnp.float32)
        d2_ref[...] = jnp.dot(hb, wc_ref[...], preferred_element_type=jnp.float32) + b_ref[...]

    return pl.pallas_call(
        body,
        grid=(N // ROW_BLK,),
        in_specs=[
            pl.BlockSpec((NC, ROW_BLK, D), lambda i: (0, i, 0)),
            pl.BlockSpec((ROW_BLK, D), lambda i: (i, 0)),
            pl.BlockSpec((D, D), lambda i: (0, 0)),
            pl.BlockSpec((D, D), lambda i: (0, 0)),
            pl.BlockSpec((1, D), lambda i: (0, 0)),
        ],
        out_specs=[
            pl.BlockSpec((ROW_BLK, D), lambda i: (i, 0)),
            pl.BlockSpec((ROW_BLK, D), lambda i: (i, 0)),
        ],
        out_shape=[
            jax.ShapeDtypeStruct((N, D), jnp.float32),
            jax.ShapeDtypeStruct((N, D), jnp.float32),
        ],
    )(p, d, wlT, wcT, bias)


def _tc_tail(p, d, owT, ob):
    def body(p_ref, d_ref, ow_ref, ob_ref, o_ref):
        hb = jnp.maximum(p_ref[0] + p_ref[1] + d_ref[...], 0.0)
        o_ref[...] = jnp.dot(hb, ow_ref[...], preferred_element_type=jnp.float32) + ob_ref[...]

    return pl.pallas_call(
        body,
        grid=(N // ROW_BLK,),
        in_specs=[
            pl.BlockSpec((NC, ROW_BLK, D), lambda i: (0, i, 0)),
            pl.BlockSpec((ROW_BLK, D), lambda i: (i, 0)),
            pl.BlockSpec((D, D), lambda i: (0, 0)),
            pl.BlockSpec((1, D), lambda i: (0, 0)),
        ],
        out_specs=pl.BlockSpec((ROW_BLK, D), lambda i: (i, 0)),
        out_shape=jax.ShapeDtypeStruct((N, D), jnp.float32),
    )(p, d, owT, ob)


def _pad_edges(ei):
    pad = jnp.arange(E_PAD - E, dtype=jnp.int32)
    src = jnp.concatenate([ei[1], pad % N])
    dst = jnp.concatenate([ei[0], N + pad % (N_PAD - N)])
    return src, dst


def kernel(x, edge_index_r0, edge_index_r1,
           l0_w0_w, l0_w0_b, l0_wl_w, l0_wl_b, l0_w1_w, l0_w1_b,
           l1_w0_w, l1_w0_b, l1_wl_w, l1_wl_b, l1_w1_w, l1_w1_b,
           out_w, out_b):
    wl1T = l1_wl_w.T
    wc1T = (l1_w0_w + l1_w1_w).T
    b1 = (l1_wl_b + l1_w0_b + l1_w1_b).reshape(1, D)
    wl0T = l0_wl_w.T
    wc0T = (l0_w0_w + l0_w1_w).T
    b0 = (l0_wl_b + l0_w0_b + l0_w1_b).reshape(1, D)
    owT = out_w.T
    ob = out_b.reshape(1, D)

    src1, dst1 = _pad_edges(edge_index_r1)
    src0, dst0 = _pad_edges(edge_index_r0)
    zeros = jnp.zeros((ROWS_PER_TILE, D), jnp.float32)

    g1, d1 = _tc_head(x, wl1T, wc1T, b1)
    p1 = _sc_segment_sum(g1, src1, dst1, zeros)
    g2, d2 = _tc_mid(p1, d1, wl0T, wc0T, b0)
    p2 = _sc_segment_sum(g2, src0, dst0, zeros)
    return _tc_tail(p2, d2, owT, ob)


# speedup vs baseline: 1.0122x; 1.0122x over previous
import functools
import jax
import jax.numpy as jnp
from jax import lax
from jax.experimental import pallas as pl
from jax.experimental.pallas import tpu as pltpu
from jax.experimental.pallas import tpu_sc as plsc

N = 10000
D = 128
E = 320000

NC = 2
NS = 16
NW = NC * NS
CHUNK = 32
E_PAD = 327680
EPW = E_PAD // NW
N_CHUNKS = EPW // CHUNK
N_PAD = 10240
ROWS_PER_TILE = N_PAD // NS

_sc_mesh = plsc.VectorSubcoreMesh(core_axis_name="c", subcore_axis_name="s")

NBUF = 8
GROUPS = N_CHUNKS // NBUF

_scratch = [pltpu.VMEM((EPW,), jnp.int32)]
for _ in range(NBUF):
    _scratch.append(pltpu.VMEM((CHUNK,), jnp.int32))
    _scratch.append(pltpu.VMEM((CHUNK, D), jnp.float32))
_scratch.append(pltpu.VMEM_SHARED((N_PAD, D), jnp.float32))
_scratch.extend([pltpu.SemaphoreType.DMA] * (3 * NBUF + 1))


@functools.partial(
    pl.kernel,
    out_type=jax.ShapeDtypeStruct((NC, N_PAD, D), jnp.float32),
    mesh=_sc_mesh,
    scratch_types=_scratch,
)
def _sc_segment_sum(g_hbm, src_hbm, dst_hbm, zeros_hbm, out_hbm, *scr):
    srcv = scr[0]
    dstb = [scr[1 + 2 * b] for b in range(NBUF)]
    rows = [scr[2 + 2 * b] for b in range(NBUF)]
    acc = scr[1 + 2 * NBUF]
    sems = scr[2 + 2 * NBUF:]
    dsem = sems[0:NBUF]
    gsem = sems[NBUF:2 * NBUF]
    ssem = sems[2 * NBUF:3 * NBUF]
    isem = sems[3 * NBUF]
    c = lax.axis_index("c")
    s = lax.axis_index("s")
    wid = s * NC + c
    base = wid * EPW

    pltpu.async_copy(src_hbm.at[pl.ds(base, EPW)], srcv, isem)
    for b in range(NBUF):
        pltpu.async_copy(dst_hbm.at[pl.ds(base + b * CHUNK, CHUNK)],
                         dstb[b], dsem[b])
    pltpu.sync_copy(zeros_hbm, acc.at[pl.ds(s * ROWS_PER_TILE, ROWS_PER_TILE)])
    plsc.subcore_barrier()

    pltpu.make_async_copy(src_hbm.at[pl.ds(base, EPW)], srcv, isem).wait()
    for b in range(NBUF):
        pltpu.async_copy(g_hbm.at[srcv.at[pl.ds(b * CHUNK, CHUNK)]],
                         rows[b], gsem[b])

    def _group(base_ch, refill):
        for b in range(NBUF):
            pltpu.make_async_copy(g_hbm.at[srcv.at[pl.ds(b * CHUNK, CHUNK)]],
                                  rows[b], gsem[b]).wait()
            pltpu.make_async_copy(dst_hbm.at[pl.ds(base, CHUNK)], dstb[b],
                                  dsem[b]).wait()
            pltpu.async_copy(rows[b], acc.at[dstb[b]], ssem[b], add=True)
        for b in range(NBUF):
            pltpu.make_async_copy(rows[b], acc.at[dstb[b]], ssem[b]).wait()
            if refill:
                nxt = base_ch + NBUF + b
                pltpu.async_copy(dst_hbm.at[pl.ds(base + nxt * CHUNK, CHUNK)],
                                 dstb[b], dsem[b])
                pltpu.async_copy(g_hbm.at[srcv.at[pl.ds(nxt * CHUNK, CHUNK)]],
                                 rows[b], gsem[b])

    @pl.loop(0, GROUPS - 1)
    def _(grp):
        _group(grp * NBUF, refill=True)

    _group((GROUPS - 1) * NBUF, refill=False)

    plsc.subcore_barrier()

    pltpu.sync_copy(acc.at[pl.ds(s * ROWS_PER_TILE, ROWS_PER_TILE)],
                    out_hbm.at[c, pl.ds(s * ROWS_PER_TILE, ROWS_PER_TILE)])


ROW_BLK = 2000


def _tc_head(h, wlT, wcT, bias):
    def body(h_ref, wl_ref, wc_ref, b_ref, g_ref, d_ref):
        hb = h_ref[...]
        g_ref[...] = jnp.dot(hb, wl_ref[...], preferred_element_type=jnp.float32)
        d_ref[...] = jnp.dot(hb, wc_ref[...], preferred_element_type=jnp.float32) + b_ref[...]

    return pl.pallas_call(
        body,
        grid=(N // ROW_BLK,),
        in_specs=[
            pl.BlockSpec((ROW_BLK, D), lambda i: (i, 0)),
            pl.BlockSpec((D, D), lambda i: (0, 0)),
            pl.BlockSpec((D, D), lambda i: (0, 0)),
            pl.BlockSpec((1, D), lambda i: (0, 0)),
        ],
        out_specs=[
            pl.BlockSpec((ROW_BLK, D), lambda i: (i, 0)),
            pl.BlockSpec((ROW_BLK, D), lambda i: (i, 0)),
        ],
        out_shape=[
            jax.ShapeDtypeStruct((N, D), jnp.float32),
            jax.ShapeDtypeStruct((N, D), jnp.float32),
        ],
    )(h, wlT, wcT, bias)


def _tc_mid(p, d, wlT, wcT, bias):
    def body(p_ref, d_ref, wl_ref, wc_ref, b_ref, g_ref, d2_ref):
        hb = jnp.maximum(p_ref[0] + p_ref[1] + d_ref[...], 0.0)
        g_ref[...] = jnp.dot(hb, wl_ref[...], preferred_element_type=jnp.float32)
        d2_ref[...] = jnp.dot(hb, wc_ref[...], preferred_element_type=jnp.float32) + b_ref[...]

    return pl.pallas_call(
        body,
        grid=(N // ROW_BLK,),
        in_specs=[
            pl.BlockSpec((NC, ROW_BLK, D), lambda i: (0, i, 0)),
            pl.BlockSpec((ROW_BLK, D), lambda i: (i, 0)),
            pl.BlockSpec((D, D), lambda i: (0, 0)),
            pl.BlockSpec((D, D), lambda i: (0, 0)),
            pl.BlockSpec((1, D), lambda i: (0, 0)),
        ],
        out_specs=[
            pl.BlockSpec((ROW_BLK, D), lambda i: (i, 0)),
            pl.BlockSpec((ROW_BLK, D), lambda i: (i, 0)),
        ],
        out_shape=[
            jax.ShapeDtypeStruct((N, D), jnp.float32),
            jax.ShapeDtypeStruct((N, D), jnp.float32),
        ],
    )(p, d, wlT, wcT, bias)


def _tc_tail(p, d, owT, ob):
    def body(p_ref, d_ref, ow_ref, ob_ref, o_ref):
        hb = jnp.maximum(p_ref[0] + p_ref[1] + d_ref[...], 0.0)
        o_ref[...] = jnp.dot(hb, ow_ref[...], preferred_element_type=jnp.float32) + ob_ref[...]

    return pl.pallas_call(
        body,
        grid=(N // ROW_BLK,),
        in_specs=[
            pl.BlockSpec((NC, ROW_BLK, D), lambda i: (0, i, 0)),
            pl.BlockSpec((ROW_BLK, D), lambda i: (i, 0)),
            pl.BlockSpec((D, D), lambda i: (0, 0)),
            pl.BlockSpec((1, D), lambda i: (0, 0)),
        ],
        out_specs=pl.BlockSpec((ROW_BLK, D), lambda i: (i, 0)),
        out_shape=jax.ShapeDtypeStruct((N, D), jnp.float32),
    )(p, d, owT, ob)


def _pad_edges(ei):
    pad = jnp.arange(E_PAD - E, dtype=jnp.int32)
    src = jnp.concatenate([ei[1], pad % N])
    dst = jnp.concatenate([ei[0], N + pad % (N_PAD - N)])
    return src, dst


def kernel(x, edge_index_r0, edge_index_r1,
           l0_w0_w, l0_w0_b, l0_wl_w, l0_wl_b, l0_w1_w, l0_w1_b,
           l1_w0_w, l1_w0_b, l1_wl_w, l1_wl_b, l1_w1_w, l1_w1_b,
           out_w, out_b):
    wl1T = l1_wl_w.T
    wc1T = (l1_w0_w + l1_w1_w).T
    b1 = (l1_wl_b + l1_w0_b + l1_w1_b).reshape(1, D)
    wl0T = l0_wl_w.T
    wc0T = (l0_w0_w + l0_w1_w).T
    b0 = (l0_wl_b + l0_w0_b + l0_w1_b).reshape(1, D)
    owT = out_w.T
    ob = out_b.reshape(1, D)

    src1, dst1 = _pad_edges(edge_index_r1)
    src0, dst0 = _pad_edges(edge_index_r0)
    zeros = jnp.zeros((ROWS_PER_TILE, D), jnp.float32)

    g1, d1 = _tc_head(x, wl1T, wc1T, b1)
    p1 = _sc_segment_sum(g1, src1, dst1, zeros)
    g2, d2 = _tc_mid(p1, d1, wl0T, wc0T, b0)
    p2 = _sc_segment_sum(g2, src0, dst0, zeros)
    return _tc_tail(p2, d2, owT, ob)


# speedup vs baseline: 1.0375x; 1.0250x over previous
import functools
import jax
import jax.numpy as jnp
from jax import lax
from jax.experimental import pallas as pl
from jax.experimental.pallas import tpu as pltpu
from jax.experimental.pallas import tpu_sc as plsc

N = 10000
D = 128
E = 320000

NC = 2
NS = 16
NW = NC * NS
CHUNK = 32
E_PAD = 327680
EPW = E_PAD // NW
N_CHUNKS = EPW // CHUNK
N_PAD = 10240
ROWS_PER_TILE = N_PAD // NS

_sc_mesh = plsc.VectorSubcoreMesh(core_axis_name="c", subcore_axis_name="s")

NBUF = 8
GROUPS = N_CHUNKS // NBUF

_scratch = [pltpu.VMEM((EPW,), jnp.int32)]
for _ in range(NBUF):
    _scratch.append(pltpu.VMEM((CHUNK,), jnp.int32))
    _scratch.append(pltpu.VMEM((CHUNK, D), jnp.float32))
_scratch.append(pltpu.VMEM_SHARED((N_PAD, D), jnp.float32))
_scratch.extend([pltpu.SemaphoreType.DMA] * (3 * NBUF + 1))


@functools.partial(
    pl.kernel,
    out_type=jax.ShapeDtypeStruct((NC, N_PAD, D), jnp.float32),
    mesh=_sc_mesh,
    scratch_types=_scratch,
)
def _sc_segment_sum(g_hbm, src_hbm, dst_hbm, zeros_hbm, out_hbm, *scr):
    srcv = scr[0]
    dstb = [scr[1 + 2 * b] for b in range(NBUF)]
    rows = [scr[2 + 2 * b] for b in range(NBUF)]
    acc = scr[1 + 2 * NBUF]
    sems = scr[2 + 2 * NBUF:]
    dsem = sems[0:NBUF]
    gsem = sems[NBUF:2 * NBUF]
    ssem = sems[2 * NBUF:3 * NBUF]
    isem = sems[3 * NBUF]
    c = lax.axis_index("c")
    s = lax.axis_index("s")
    wid = s * NC + c
    base = wid * EPW

    pltpu.async_copy(src_hbm.at[pl.ds(base, EPW)], srcv, isem)
    for b in range(NBUF):
        pltpu.async_copy(dst_hbm.at[pl.ds(base + b * CHUNK, CHUNK)],
                         dstb[b], dsem[b])
    pltpu.sync_copy(zeros_hbm, rows[0])
    for k in range(ROWS_PER_TILE // CHUNK):
        b = k % NBUF
        if k >= NBUF:
            pltpu.make_async_copy(
                rows[0], acc.at[pl.ds(s * ROWS_PER_TILE, CHUNK)], ssem[b]).wait()
        pltpu.async_copy(rows[0],
                         acc.at[pl.ds(s * ROWS_PER_TILE + k * CHUNK, CHUNK)],
                         ssem[b])
    for b in range(NBUF):
        pltpu.make_async_copy(
            rows[0], acc.at[pl.ds(s * ROWS_PER_TILE, CHUNK)], ssem[b]).wait()
    plsc.subcore_barrier()

    pltpu.make_async_copy(src_hbm.at[pl.ds(base, EPW)], srcv, isem).wait()
    for b in range(NBUF):
        pltpu.async_copy(g_hbm.at[srcv.at[pl.ds(b * CHUNK, CHUNK)]],
                         rows[b], gsem[b])

    def _group(base_ch, refill):
        for b in range(NBUF):
            pltpu.make_async_copy(g_hbm.at[srcv.at[pl.ds(b * CHUNK, CHUNK)]],
                                  rows[b], gsem[b]).wait()
            pltpu.make_async_copy(dst_hbm.at[pl.ds(base, CHUNK)], dstb[b],
                                  dsem[b]).wait()
            pltpu.async_copy(rows[b], acc.at[dstb[b]], ssem[b], add=True)
        for b in range(NBUF):
            pltpu.make_async_copy(rows[b], acc.at[dstb[b]], ssem[b]).wait()
            if refill:
                nxt = base_ch + NBUF + b
                pltpu.async_copy(dst_hbm.at[pl.ds(base + nxt * CHUNK, CHUNK)],
                                 dstb[b], dsem[b])
                pltpu.async_copy(g_hbm.at[srcv.at[pl.ds(nxt * CHUNK, CHUNK)]],
                                 rows[b], gsem[b])

    @pl.loop(0, GROUPS - 1)
    def _(grp):
        _group(grp * NBUF, refill=True)

    _group((GROUPS - 1) * NBUF, refill=False)

    plsc.subcore_barrier()

    pltpu.sync_copy(acc.at[pl.ds(s * ROWS_PER_TILE, ROWS_PER_TILE)],
                    out_hbm.at[c, pl.ds(s * ROWS_PER_TILE, ROWS_PER_TILE)])


ROW_BLK = 2000


def _tc_head(h, wlT, wcT, bias):
    def body(h_ref, wl_ref, wc_ref, b_ref, g_ref, d_ref):
        hb = h_ref[...]
        g_ref[...] = jnp.dot(hb, wl_ref[...], preferred_element_type=jnp.float32)
        d_ref[...] = jnp.dot(hb, wc_ref[...], preferred_element_type=jnp.float32) + b_ref[...]

    return pl.pallas_call(
        body,
        grid=(N // ROW_BLK,),
        in_specs=[
            pl.BlockSpec((ROW_BLK, D), lambda i: (i, 0)),
            pl.BlockSpec((D, D), lambda i: (0, 0)),
            pl.BlockSpec((D, D), lambda i: (0, 0)),
            pl.BlockSpec((1, D), lambda i: (0, 0)),
        ],
        out_specs=[
            pl.BlockSpec((ROW_BLK, D), lambda i: (i, 0)),
            pl.BlockSpec((ROW_BLK, D), lambda i: (i, 0)),
        ],
        out_shape=[
            jax.ShapeDtypeStruct((N, D), jnp.float32),
            jax.ShapeDtypeStruct((N, D), jnp.float32),
        ],
    )(h, wlT, wcT, bias)


def _tc_mid(p, d, wlT, wcT, bias):
    def body(p_ref, d_ref, wl_ref, wc_ref, b_ref, g_ref, d2_ref):
        hb = jnp.maximum(p_ref[0] + p_ref[1] + d_ref[...], 0.0)
        g_ref[...] = jnp.dot(hb, wl_ref[...], preferred_element_type=jnp.float32)
        d2_ref[...] = jnp.dot(hb, wc_ref[...], preferred_element_type=jnp.float32) + b_ref[...]

    return pl.pallas_call(
        body,
        grid=(N // ROW_BLK,),
        in_specs=[
            pl.BlockSpec((NC, ROW_BLK, D), lambda i: (0, i, 0)),
            pl.BlockSpec((ROW_BLK, D), lambda i: (i, 0)),
            pl.BlockSpec((D, D), lambda i: (0, 0)),
            pl.BlockSpec((D, D), lambda i: (0, 0)),
            pl.BlockSpec((1, D), lambda i: (0, 0)),
        ],
        out_specs=[
            pl.BlockSpec((ROW_BLK, D), lambda i: (i, 0)),
            pl.BlockSpec((ROW_BLK, D), lambda i: (i, 0)),
        ],
        out_shape=[
            jax.ShapeDtypeStruct((N, D), jnp.float32),
            jax.ShapeDtypeStruct((N, D), jnp.float32),
        ],
    )(p, d, wlT, wcT, bias)


def _tc_tail(p, d, owT, ob):
    def body(p_ref, d_ref, ow_ref, ob_ref, o_ref):
        hb = jnp.maximum(p_ref[0] + p_ref[1] + d_ref[...], 0.0)
        o_ref[...] = jnp.dot(hb, ow_ref[...], preferred_element_type=jnp.float32) + ob_ref[...]

    return pl.pallas_call(
        body,
        grid=(N // ROW_BLK,),
        in_specs=[
            pl.BlockSpec((NC, ROW_BLK, D), lambda i: (0, i, 0)),
            pl.BlockSpec((ROW_BLK, D), lambda i: (i, 0)),
            pl.BlockSpec((D, D), lambda i: (0, 0)),
            pl.BlockSpec((1, D), lambda i: (0, 0)),
        ],
        out_specs=pl.BlockSpec((ROW_BLK, D), lambda i: (i, 0)),
        out_shape=jax.ShapeDtypeStruct((N, D), jnp.float32),
    )(p, d, owT, ob)


def _pad_edges(ei):
    pad = jnp.arange(E_PAD - E, dtype=jnp.int32)
    src = jnp.concatenate([ei[1], pad % N])
    dst = jnp.concatenate([ei[0], N + pad % (N_PAD - N)])
    return src, dst


def kernel(x, edge_index_r0, edge_index_r1,
           l0_w0_w, l0_w0_b, l0_wl_w, l0_wl_b, l0_w1_w, l0_w1_b,
           l1_w0_w, l1_w0_b, l1_wl_w, l1_wl_b, l1_w1_w, l1_w1_b,
           out_w, out_b):
    wl1T = l1_wl_w.T
    wc1T = (l1_w0_w + l1_w1_w).T
    b1 = (l1_wl_b + l1_w0_b + l1_w1_b).reshape(1, D)
    wl0T = l0_wl_w.T
    wc0T = (l0_w0_w + l0_w1_w).T
    b0 = (l0_wl_b + l0_w0_b + l0_w1_b).reshape(1, D)
    owT = out_w.T
    ob = out_b.reshape(1, D)

    src1, dst1 = _pad_edges(edge_index_r1)
    src0, dst0 = _pad_edges(edge_index_r0)
    zeros = jnp.zeros((CHUNK, D), jnp.float32)

    g1, d1 = _tc_head(x, wl1T, wc1T, b1)
    p1 = _sc_segment_sum(g1, src1, dst1, zeros)
    g2, d2 = _tc_mid(p1, d1, wl0T, wc0T, b0)
    p2 = _sc_segment_sum(g2, src0, dst0, zeros)
    return _tc_tail(p2, d2, owT, ob)
